# initial kernel scaffold (unmeasured)
import jax
import jax.numpy as jnp
from jax import lax
from jax.experimental import pallas as pl
from jax.experimental.pallas import tpu as pltpu


def kernel(
    x,
):
    def body(*refs):
        pass

    out_shape = jax.ShapeDtypeStruct(..., jnp.float32)
    return pl.pallas_call(body, out_shape=out_shape)(...)


# baseline (device time: 29527 ns/iter reference)
import jax
import jax.numpy as jnp
from jax import lax
from jax.experimental import pallas as pl
from jax.experimental.pallas import tpu as pltpu


def kernel(x):
    m, n = x.shape

    def body(x_ref, out_ref, recv_ref, send_sem, recv_sem):
        my_x = lax.axis_index("x")
        my_y = lax.axis_index("y")
        my_z = lax.axis_index("z")
        peer = (1 - my_x, my_y, my_z)

        barrier_sem = pltpu.get_barrier_semaphore()
        pl.semaphore_signal(
            barrier_sem, inc=1, device_id=peer,
            device_id_type=pl.DeviceIdType.MESH,
        )
        pl.semaphore_wait(barrier_sem, 1)

        rdma = pltpu.make_async_remote_copy(
            src_ref=x_ref,
            dst_ref=recv_ref,
            send_sem=send_sem,
            recv_sem=recv_sem,
            device_id=peer,
            device_id_type=pl.DeviceIdType.MESH,
        )
        rdma.start()
        rdma.wait()

        out_ref[...] = x_ref[...] + recv_ref[...]

    return pl.pallas_call(
        body,
        out_shape=jax.ShapeDtypeStruct((m, n), jnp.float32),
        in_specs=[pl.BlockSpec(memory_space=pltpu.VMEM)],
        out_specs=pl.BlockSpec(memory_space=pltpu.VMEM),
        scratch_shapes=[
            pltpu.VMEM((m, n), jnp.float32),
            pltpu.SemaphoreType.DMA,
            pltpu.SemaphoreType.DMA,
        ],
        compiler_params=pltpu.CompilerParams(collective_id=0),
    )(x)


# device time: 19273 ns/iter; 1.5320x vs baseline; 1.5320x over previous
import jax
import jax.numpy as jnp
from jax import lax
from jax.experimental import pallas as pl
from jax.experimental.pallas import tpu as pltpu

C = 4
QR = 256
R = QR // C
MESH = pl.DeviceIdType.MESH


def kernel(x):
    m, n = x.shape

    def body(x_ref, out_ref, xrecv, sx_s, sx_r, sy_s, sy_r, sz_s, sz_r,
             sfy_s, sfy_r, sfz_s, sfz_r):
        my_x = lax.axis_index("x")
        my_y = lax.axis_index("y")
        my_z = lax.axis_index("z")
        xpeer = (1 - my_x, my_y, my_z)
        ypeer = (my_x, 1 - my_y, my_z)
        zpeer = (my_x, my_y, 1 - my_z)
        q = 2 * my_y + my_z
        qy = 2 * (1 - my_y) + my_z
        qz = 2 * my_y + (1 - my_z)
        qd = 2 * (1 - my_y) + (1 - my_z)

        barrier_sem = pltpu.get_barrier_semaphore()
        for p in (xpeer, ypeer, zpeer):
            pl.semaphore_signal(barrier_sem, inc=1, device_id=p,
                                device_id_type=MESH)
        pl.semaphore_wait(barrier_sem, 3)

        xd = []
        for c in range(C):
            d = pltpu.make_async_remote_copy(
                src_ref=x_ref.at[pl.ds(q * QR + c * R, R)],
                dst_ref=xrecv.at[pl.ds(c * R, R)],
                send_sem=sx_s.at[c], recv_sem=sx_r.at[c],
                device_id=xpeer, device_id_type=MESH)
            d.start()
            xd.append(d)

        yd, zd = [], []
        for c in range(C):
            xd[c].wait()
            rows = pl.ds(q * QR + c * R, R)
            out_ref[rows, :] = x_ref[rows, :] + xrecv[pl.ds(c * R, R), :]
            dy = pltpu.make_async_remote_copy(
                src_ref=out_ref.at[rows], dst_ref=out_ref.at[rows],
                send_sem=sy_s.at[c], recv_sem=sy_r.at[c],
                device_id=ypeer, device_id_type=MESH)
            dy.start()
            yd.append(dy)
            dz = pltpu.make_async_remote_copy(
                src_ref=out_ref.at[rows], dst_ref=out_ref.at[rows],
                send_sem=sz_s.at[c], recv_sem=sz_r.at[c],
                device_id=zpeer, device_id_type=MESH)
            dz.start()
            zd.append(dz)

        H = R // 2
        fzd, fyd = [], []
        for c in range(C):
            ry = pltpu.make_async_remote_copy(
                src_ref=out_ref.at[pl.ds(qy * QR + c * R, R)],
                dst_ref=out_ref.at[pl.ds(qy * QR + c * R, R)],
                send_sem=sy_s.at[c], recv_sem=sy_r.at[c],
                device_id=ypeer, device_id_type=MESH)
            ry.wait_recv()
            fz = pltpu.make_async_remote_copy(
                src_ref=out_ref.at[pl.ds(qy * QR + c * R, H)],
                dst_ref=out_ref.at[pl.ds(qy * QR + c * R, H)],
                send_sem=sfz_s.at[c], recv_sem=sfz_r.at[c],
                device_id=zpeer, device_id_type=MESH)
            fz.start()
            fzd.append(fz)

            rz = pltpu.make_async_remote_copy(
                src_ref=out_ref.at[pl.ds(qz * QR + c * R, R)],
                dst_ref=out_ref.at[pl.ds(qz * QR + c * R, R)],
                send_sem=sz_s.at[c], recv_sem=sz_r.at[c],
                device_id=zpeer, device_id_type=MESH)
            rz.wait_recv()
            fy = pltpu.make_async_remote_copy(
                src_ref=out_ref.at[pl.ds(qz * QR + c * R + H, H)],
                dst_ref=out_ref.at[pl.ds(qz * QR + c * R + H, H)],
                send_sem=sfy_s.at[c], recv_sem=sfy_r.at[c],
                device_id=ypeer, device_id_type=MESH)
            fy.start()
            fyd.append(fy)

        for c in range(C):
            rfz = pltpu.make_async_remote_copy(
                src_ref=out_ref.at[pl.ds(qd * QR + c * R, H)],
                dst_ref=out_ref.at[pl.ds(qd * QR + c * R, H)],
                send_sem=sfz_s.at[c], recv_sem=sfz_r.at[c],
                device_id=zpeer, device_id_type=MESH)
            rfz.wait_recv()
            rfy = pltpu.make_async_remote_copy(
                src_ref=out_ref.at[pl.ds(qd * QR + c * R + H, H)],
                dst_ref=out_ref.at[pl.ds(qd * QR + c * R + H, H)],
                send_sem=sfy_s.at[c], recv_sem=sfy_r.at[c],
                device_id=ypeer, device_id_type=MESH)
            rfy.wait_recv()
        for c in range(C):
            yd[c].wait_send()
            zd[c].wait_send()
            fzd[c].wait_send()
            fyd[c].wait_send()

    return pl.pallas_call(
        body,
        out_shape=jax.ShapeDtypeStruct((m, n), jnp.float32),
        in_specs=[pl.BlockSpec(memory_space=pltpu.VMEM)],
        out_specs=pl.BlockSpec(memory_space=pltpu.VMEM),
        scratch_shapes=[
            pltpu.VMEM((QR, n), jnp.float32),
            pltpu.SemaphoreType.DMA((C,)), pltpu.SemaphoreType.DMA((C,)),
            pltpu.SemaphoreType.DMA((C,)), pltpu.SemaphoreType.DMA((C,)),
            pltpu.SemaphoreType.DMA((C,)), pltpu.SemaphoreType.DMA((C,)),
            pltpu.SemaphoreType.DMA((C,)), pltpu.SemaphoreType.DMA((C,)),
            pltpu.SemaphoreType.DMA((C,)), pltpu.SemaphoreType.DMA((C,)),
        ],
        compiler_params=pltpu.CompilerParams(collective_id=0),
    )(x)
